# Initial kernel scaffold; baseline (speedup 1.0000x reference)
import jax
import jax.numpy as jnp
from jax.experimental import pallas as pl


def kernel(x, edge_index, W1, att_src1, att_dst1, b1, W2, att_src2, att_dst2, b2):
    raise NotImplementedError("write your pallas kernel here")


# speedup vs baseline: 2.0057x; 2.0057x over previous
import functools

import jax
import jax.numpy as jnp
from jax import lax
from jax.experimental import pallas as pl
from jax.experimental.pallas import tpu as pltpu
from jax.experimental.pallas import tpu_sc as plsc

NN = 10000
EE = 160000
D_IN = 128
H1 = 64
C1 = 64
D1 = H1 * C1
H2 = 5

NTILES = 32
EPRIME = EE + NN
EPCAP = EE + NN + 7 * NN
SRCWIN = 1024
EPAD = EPCAP + SRCWIN + 64
NODEWIN = 128
NPADROWS = NN + NODEWIN + 64

_MESH = plsc.VectorSubcoreMesh(core_axis_name="c", subcore_axis_name="s")


_BLK_A = 400


def _mm1_body(x_ref, w_ref, ast_ref, adt_ref, xh_ref, aa_ref):
    xb = x_ref[...]
    xh = jnp.dot(xb, w_ref[...], preferred_element_type=jnp.float32)
    xh_ref[...] = xh
    xr = xh.reshape(_BLK_A, C1, H1)
    asv = jnp.sum(xr * ast_ref[...][None], axis=1)
    adv = jnp.sum(xr * adt_ref[...][None], axis=1)
    aa_ref[...] = jnp.concatenate([asv, adv], axis=1)


def _mm1(x, w1p, a_src_t, a_dst_t):
    grid = (NN // _BLK_A,)
    return pl.pallas_call(
        _mm1_body,
        grid=grid,
        in_specs=[
            pl.BlockSpec((_BLK_A, D_IN), lambda i: (i, jnp.int32(0))),
            pl.BlockSpec((D_IN, D1), lambda i: (jnp.int32(0), jnp.int32(0))),
            pl.BlockSpec((C1, H1), lambda i: (jnp.int32(0), jnp.int32(0))),
            pl.BlockSpec((C1, H1), lambda i: (jnp.int32(0), jnp.int32(0))),
        ],
        out_specs=[
            pl.BlockSpec((_BLK_A, D1), lambda i: (i, jnp.int32(0))),
            pl.BlockSpec((_BLK_A, 2 * H1), lambda i: (i, jnp.int32(0))),
        ],
        out_shape=[
            jax.ShapeDtypeStruct((NN, D1), jnp.float32),
            jax.ShapeDtypeStruct((NN, 2 * H1), jnp.float32),
        ],
    )(x, w1p, a_src_t, a_dst_t)


@functools.partial(
    pl.kernel,
    mesh=_MESH,
    out_type=(
        jax.ShapeDtypeStruct((NN, D1), jnp.float32),
        jax.ShapeDtypeStruct((NN, H1), jnp.float32),
    ),
    scratch_types=[
        pltpu.VMEM((56,), jnp.int32),
        pltpu.VMEM((NODEWIN + 32,), jnp.int32),
        pltpu.VMEM((NODEWIN + 32,), jnp.int32),
        pltpu.VMEM((NODEWIN, 2 * H1), jnp.float32),
        pltpu.VMEM((SRCWIN + 64,), jnp.int32),
        pltpu.VMEM((8, D1), jnp.float32),
        pltpu.VMEM((8, 2 * H1), jnp.float32),
        pltpu.VMEM((D1,), jnp.float32),
        pltpu.VMEM((H1,), jnp.float32),
        pltpu.SemaphoreType.DMA,
        pltpu.SemaphoreType.DMA,
    ],
)
def _agg1(xh_hbm, aap_hbm, srcp_hbm, prow_hbm, deg_hbm, tb_hbm,
          acc_hbm, den_hbm,
          tbbuf, prowbuf, degbuf, adwin, srcbuf, rowbuf, asbuf, accbuf, dnbuf,
          sem0, sem1):
    def _sread(ref, idx):
        return ref[pl.ds(idx, 16)][0]

    wid = lax.axis_index("s") * jnp.int32(2) + lax.axis_index("c")
    pltpu.sync_copy(tb_hbm, tbbuf)
    n0 = _sread(tbbuf, wid)
    n1 = _sread(tbbuf, wid + jnp.int32(1))

    def refill_nodes(base):
        base = pl.multiple_of(base, 8)
        pltpu.sync_copy(prow_hbm.at[pl.ds(base, NODEWIN + 32)], prowbuf)
        pltpu.sync_copy(deg_hbm.at[pl.ds(base, NODEWIN + 32)], degbuf)
        pltpu.sync_copy(aap_hbm.at[pl.ds(base, NODEWIN)], adwin)

    nwb0 = (n0 // jnp.int32(8)) * jnp.int32(8)

    @pl.when(n1 > n0)
    def _():
        refill_nodes(nwb0)

    zero16 = jnp.zeros((16,), jnp.float32)
    for hg in range(4):
        dnbuf[pl.ds(hg * 16, 16)] = zero16
    for c in range(0, D1, 16):
        accbuf[pl.ds(c, 16)] = zero16

    def node_body(i, carry):
        nwb, swb = carry
        need_n = (i - nwb) >= jnp.int32(NODEWIN)

        @pl.when(need_n)
        def _():
            refill_nodes((i // jnp.int32(8)) * jnp.int32(8))

        nwb = jnp.where(need_n, (i // jnp.int32(8)) * jnp.int32(8), nwb)
        irel = i - nwb
        pcur = _sread(prowbuf, irel)
        pnext = _sread(prowbuf, irel + jnp.int32(1))
        deg = _sread(degbuf, irel)
        ngrp = (pnext - pcur) // jnp.int32(8)

        adrow = [adwin[irel, pl.ds(H1 + hg * 16, 16)] for hg in range(4)]

        def grp_body(g, swb):
            cur = pl.multiple_of(pcur + g * jnp.int32(8), 8)
            need_s = (cur - swb) + jnp.int32(8) > jnp.int32(SRCWIN)

            @pl.when(need_s)
            def _():
                pltpu.sync_copy(srcp_hbm.at[pl.ds(cur, SRCWIN)],
                                srcbuf.at[pl.ds(0, SRCWIN)])

            swb = jnp.where(need_s, cur, swb)
            rel = pl.multiple_of(cur - swb, 8)
            idxref = srcbuf.at[pl.ds(rel, 8)]
            cp0 = pltpu.async_copy(xh_hbm.at[idxref], rowbuf, sem0)
            cp1 = pltpu.async_copy(aap_hbm.at[idxref], asbuf, sem1)
            cp0.wait()
            cp1.wait()

            pvecs = []
            for j in range(8):
                w = jnp.where(g * jnp.int32(8) + jnp.int32(j) < deg, jnp.float32(1.0), jnp.float32(0.0))
                pj = []
                for hg in range(4):
                    e = asbuf[j, pl.ds(hg * 16, 16)] + adrow[hg]
                    e = jnp.maximum(e, 0.2 * e)
                    p = jnp.exp(e) * w
                    off = hg * 16
                    dnbuf[pl.ds(off, 16)] = dnbuf[pl.ds(off, 16)] + p
                    pj.append(p)
                pvecs.append(pj)

            def mac_body(c, z):
                base = c * jnp.int32(H1)
                for hg in range(4):
                    off = base + jnp.int32(hg * 16)
                    a = accbuf[pl.ds(off, 16)]
                    for j in range(8):
                        a = a + rowbuf[j, pl.ds(off, 16)] * pvecs[j][hg]
                    accbuf[pl.ds(off, 16)] = a
                return z

            lax.fori_loop(jnp.int32(0), jnp.int32(C1), mac_body, jnp.int32(0))
            return swb

        swb = lax.fori_loop(jnp.int32(0), ngrp, grp_body, swb)

        pltpu.sync_copy(accbuf, acc_hbm.at[i])
        pltpu.sync_copy(dnbuf, den_hbm.at[i])
        for hg in range(4):
            dnbuf[pl.ds(hg * 16, 16)] = zero16
        for c in range(0, D1, 16):
            accbuf[pl.ds(c, 16)] = zero16
        return (nwb, swb)

    lax.fori_loop(n0, n1, node_body, (nwb0, jnp.int32(-(1 << 28))))


_BLK_C = 200


def _mm2_body(acc_ref, den_ref, w2_ref, b1_ref, out_ref):
    dinv = 1.0 / den_ref[...]
    h = acc_ref[...].reshape(_BLK_C, C1, H1) * dinv[:, None, :]
    h = h + b1_ref[...][None]
    h = jnp.where(h > 0, h, jnp.exp(h) - 1.0)
    out_ref[...] = jnp.dot(h.reshape(_BLK_C, D1), w2_ref[...],
                           preferred_element_type=jnp.float32)


def _mm2(acc, den, w2pp, b1p):
    grid = (NN // _BLK_C,)
    return pl.pallas_call(
        _mm2_body,
        grid=grid,
        in_specs=[
            pl.BlockSpec((_BLK_C, D1), lambda i: (i, jnp.int32(0))),
            pl.BlockSpec((_BLK_C, H1), lambda i: (i, jnp.int32(0))),
            pl.BlockSpec((D1, 128), lambda i: (jnp.int32(0), jnp.int32(0))),
            pl.BlockSpec((C1, H1), lambda i: (jnp.int32(0), jnp.int32(0))),
        ],
        out_specs=pl.BlockSpec((_BLK_C, 128), lambda i: (i, jnp.int32(0))),
        out_shape=jax.ShapeDtypeStruct((NN, 128), jnp.float32),
    )(acc, den, w2pp, b1p)


@functools.partial(
    pl.kernel,
    mesh=_MESH,
    out_type=jax.ShapeDtypeStruct((NTILES, 16), jnp.float32),
    scratch_types=[
        pltpu.VMEM((56,), jnp.int32),
        pltpu.VMEM((NODEWIN + 32,), jnp.int32),
        pltpu.VMEM((NODEWIN + 32,), jnp.int32),
        pltpu.VMEM((NODEWIN, 128), jnp.float32),
        pltpu.VMEM((SRCWIN + 64,), jnp.int32),
        pltpu.VMEM((8, 128), jnp.float32),
        pltpu.VMEM((16,), jnp.float32),
        pltpu.VMEM((16,), jnp.float32),
        pltpu.VMEM((16,), jnp.float32),
        pltpu.SemaphoreType.DMA,
    ],
)
def _agg2(xw_hbm, srcp_hbm, prow_hbm, deg_hbm, tb_hbm, a2s_hbm, a2d_hbm,
          out_hbm,
          tbbuf, prowbuf, degbuf, xwwin, srcbuf, rowbuf, a2sbuf, a2dbuf,
          tsbuf, sem0):
    def _sread(ref, idx):
        return ref[pl.ds(idx, 16)][0]

    wid = lax.axis_index("s") * jnp.int32(2) + lax.axis_index("c")
    pltpu.sync_copy(tb_hbm, tbbuf)
    pltpu.sync_copy(a2s_hbm, a2sbuf)
    pltpu.sync_copy(a2d_hbm, a2dbuf)
    n0 = _sread(tbbuf, wid)
    n1 = _sread(tbbuf, wid + jnp.int32(1))
    a2s = a2sbuf[...]
    a2d = a2dbuf[...]
    colmask = lax.iota(jnp.int32, 16) < H2

    def refill_nodes(base):
        base = pl.multiple_of(base, 8)
        pltpu.sync_copy(prow_hbm.at[pl.ds(base, NODEWIN + 32)], prowbuf)
        pltpu.sync_copy(deg_hbm.at[pl.ds(base, NODEWIN + 32)], degbuf)
        pltpu.sync_copy(xw_hbm.at[pl.ds(base, NODEWIN)], xwwin)

    nwb0 = (n0 // jnp.int32(8)) * jnp.int32(8)

    @pl.when(n1 > n0)
    def _():
        refill_nodes(nwb0)

    tsbuf[...] = jnp.zeros((16,), jnp.float32)

    def node_body(i, carry):
        nwb, swb = carry
        need_n = (i - nwb) >= jnp.int32(NODEWIN)

        @pl.when(need_n)
        def _():
            refill_nodes((i // jnp.int32(8)) * jnp.int32(8))

        nwb = jnp.where(need_n, (i // jnp.int32(8)) * jnp.int32(8), nwb)
        irel = i - nwb
        pcur = _sread(prowbuf, irel)
        pnext = _sread(prowbuf, irel + jnp.int32(1))
        deg = _sread(degbuf, irel)
        ngrp = (pnext - pcur) // jnp.int32(8)
        adrow = xwwin[irel, pl.ds(0, 16)] * a2d

        def grp_body(g, carry2):
            swb, acc, dsum = carry2
            cur = pl.multiple_of(pcur + g * jnp.int32(8), 8)
            need_s = (cur - swb) + jnp.int32(8) > jnp.int32(SRCWIN)

            @pl.when(need_s)
            def _():
                pltpu.sync_copy(srcp_hbm.at[pl.ds(cur, SRCWIN)],
                                srcbuf.at[pl.ds(0, SRCWIN)])

            swb = jnp.where(need_s, cur, swb)
            rel = pl.multiple_of(cur - swb, 8)
            pltpu.async_copy(xw_hbm.at[srcbuf.at[pl.ds(rel, 8)]], rowbuf,
                             sem0).wait()
            for j in range(8):
                w = jnp.where(g * jnp.int32(8) + jnp.int32(j) < deg, jnp.float32(1.0), jnp.float32(0.0))
                srow = rowbuf[j, pl.ds(0, 16)]
                e = srow * a2s + adrow
                e = jnp.maximum(e, 0.2 * e)
                p = jnp.where(colmask, jnp.exp(e) * w, 0.0)
                acc = acc + p * srow
                dsum = dsum + p
            return (swb, acc, dsum)

        z16 = jnp.zeros((16,), jnp.float32)
        swb, acc, dsum = lax.fori_loop(jnp.int32(0), ngrp, grp_body, (swb, z16, z16))
        contrib = jnp.where(colmask, acc / dsum, 0.0)
        tsbuf[...] = tsbuf[...] + contrib
        return (nwb, swb)

    lax.fori_loop(n0, n1, node_body, (nwb0, jnp.int32(-(1 << 28))))
    pltpu.sync_copy(tsbuf, out_hbm.at[wid])


def _fin_body(ts_ref, b2_ref, out_ref):
    s = jnp.sum(ts_ref[...]) / (H2 * NN) + jnp.sum(b2_ref[...])
    out_ref[...] = jnp.broadcast_to(jnp.tanh(s), (1, 1))


def _fin(tilesums, b2):
    return pl.pallas_call(
        _fin_body,
        grid=(1,),
        in_specs=[
            pl.BlockSpec((NTILES, 16), lambda i: (jnp.int32(0), jnp.int32(0))),
            pl.BlockSpec((1, 1), lambda i: (jnp.int32(0), jnp.int32(0))),
        ],
        out_specs=pl.BlockSpec((1, 1), lambda i: (jnp.int32(0), jnp.int32(0))),
        out_shape=jax.ShapeDtypeStruct((1, 1), jnp.float32),
    )(tilesums, b2)


def kernel(x, edge_index, W1, att_src1, att_dst1, b1, W2, att_src2, att_dst2, b2):
    f32 = jnp.float32
    i32 = jnp.int32
    x = x.astype(f32)
    W1 = W1.astype(f32)
    W2 = W2.astype(f32)

    w1r = W1.reshape(D_IN, H1, C1)
    w1p = jnp.transpose(w1r, (0, 2, 1)).reshape(D_IN, D1)
    a_src_t = jnp.transpose(att_src1[0].astype(f32))
    a_dst_t = jnp.transpose(att_dst1[0].astype(f32))
    b1p = jnp.transpose(b1.astype(f32).reshape(H1, C1))
    w2p = jnp.transpose(W2.reshape(H1, C1, H2), (1, 0, 2)).reshape(D1, H2)
    w2pp = jnp.zeros((D1, 128), f32).at[:, :H2].set(w2p)
    a2s = jnp.zeros((16,), f32).at[:H2].set(att_src2[0, :, 0].astype(f32))
    a2d = jnp.zeros((16,), f32).at[:H2].set(att_dst2[0, :, 0].astype(f32))
    b2r = b2.astype(f32).reshape(1, 1)

    srce = edge_index[0].astype(i32)
    dste = edge_index[1].astype(i32)
    loop = jnp.arange(NN, dtype=i32)
    src_all = jnp.concatenate([srce, loop])
    dst_all = jnp.concatenate([dste, loop])
    order = jnp.argsort(dst_all)
    src_s = src_all[order]
    dst_s = dst_all[order]
    rowptr = jnp.searchsorted(dst_s, jnp.arange(NN + 1, dtype=i32)).astype(i32)
    deg = rowptr[1:] - rowptr[:-1]
    pdeg = ((deg + 7) // 8) * 8
    prow = jnp.concatenate(
        [jnp.zeros((1,), i32), jnp.cumsum(pdeg).astype(i32)])
    firstsrc = src_s[rowptr[:-1]]
    nid = jnp.clip(
        jnp.searchsorted(prow, jnp.arange(EPCAP, dtype=i32), side="right") - 1,
        0, NN - 1)
    pos = prow[dst_s] + (jnp.arange(EPRIME, dtype=i32) - rowptr[dst_s])
    srcp = firstsrc[nid].at[pos].set(src_s)
    srcp = jnp.concatenate([srcp, jnp.zeros((EPAD - EPCAP,), i32)])
    targets = (jnp.arange(NTILES + 1, dtype=i32) * prow[NN]) // NTILES
    tb = jnp.searchsorted(prow, targets).astype(i32)
    tb = tb.at[0].set(0).at[NTILES].set(NN)
    tb = jnp.concatenate([tb, jnp.zeros((23,), i32)])
    prow_p = jnp.concatenate(
        [prow, jnp.full((NPADROWS - NN - 1,), prow[NN], i32)])
    deg_p = jnp.concatenate([deg, jnp.zeros((NPADROWS - NN,), i32)])

    xh1, asad = _mm1(x, w1p, a_src_t, a_dst_t)
    aap = jnp.concatenate([asad, jnp.zeros((NPADROWS - NN, 2 * H1), f32)])

    acc, den = _agg1(xh1, aap, srcp, prow_p, deg_p, tb)

    xh2 = _mm2(acc, den, w2pp, b1p)
    xh2p = jnp.concatenate([xh2, jnp.zeros((NPADROWS - NN, 128), f32)])

    tilesums = _agg2(xh2p, srcp, prow_p, deg_p, tb, a2s, a2d)

    out = _fin(tilesums, b2r)
    return out.reshape(1)


# speedup vs baseline: 9.7715x; 4.8718x over previous
import functools

import jax
import jax.numpy as jnp
from jax import lax
from jax.experimental import pallas as pl
from jax.experimental.pallas import tpu as pltpu
from jax.experimental.pallas import tpu_sc as plsc

NN = 10000
EE = 160000
D_IN = 128
H1 = 64
C1 = 64
D1 = H1 * C1
H2 = 5

NTILES = 32
EPRIME = EE + NN
SRCWIN = 1024
EPAD = EPRIME + SRCWIN + 64
NODEWIN = 128
NPADROWS = NN + NODEWIN + 64

_MESH = plsc.VectorSubcoreMesh(core_axis_name="c", subcore_axis_name="s")


_BLK_A = 400


def _mm1_body(x_ref, w_ref, ast_ref, adt_ref, xh_ref, aa_ref):
    xb = x_ref[...]
    xh = jnp.dot(xb, w_ref[...], preferred_element_type=jnp.float32)
    xh_ref[...] = xh
    xr = xh.reshape(_BLK_A, C1, H1)
    asv = jnp.sum(xr * ast_ref[...][None], axis=1)
    adv = jnp.sum(xr * adt_ref[...][None], axis=1)
    aa_ref[...] = jnp.concatenate([asv, adv], axis=1)


def _mm1(x, w1p, a_src_t, a_dst_t):
    grid = (NN // _BLK_A,)
    return pl.pallas_call(
        _mm1_body,
        grid=grid,
        in_specs=[
            pl.BlockSpec((_BLK_A, D_IN), lambda i: (i, jnp.int32(0))),
            pl.BlockSpec((D_IN, D1), lambda i: (jnp.int32(0), jnp.int32(0))),
            pl.BlockSpec((C1, H1), lambda i: (jnp.int32(0), jnp.int32(0))),
            pl.BlockSpec((C1, H1), lambda i: (jnp.int32(0), jnp.int32(0))),
        ],
        out_specs=[
            pl.BlockSpec((_BLK_A, D1), lambda i: (i, jnp.int32(0))),
            pl.BlockSpec((_BLK_A, 2 * H1), lambda i: (i, jnp.int32(0))),
        ],
        out_shape=[
            jax.ShapeDtypeStruct((NN, D1), jnp.float32),
            jax.ShapeDtypeStruct((NN, 2 * H1), jnp.float32),
        ],
    )(x, w1p, a_src_t, a_dst_t)


@functools.partial(
    pl.kernel,
    mesh=_MESH,
    out_type=(
        jax.ShapeDtypeStruct((NN, D1), jnp.float32),
        jax.ShapeDtypeStruct((NN, H1), jnp.float32),
    ),
    scratch_types=[
        pltpu.VMEM((56,), jnp.int32),
        pltpu.VMEM((NODEWIN + 32,), jnp.int32),
        pltpu.VMEM((NODEWIN, 2 * H1), jnp.float32),
        pltpu.VMEM((SRCWIN + 64,), jnp.int32),
        pltpu.VMEM((8, D1), jnp.float32),
        pltpu.VMEM((8, 2 * H1), jnp.float32),
        pltpu.VMEM((D1,), jnp.float32),
        pltpu.VMEM((H1,), jnp.float32),
        pltpu.SemaphoreType.DMA,
        pltpu.SemaphoreType.DMA,
    ],
)
def _agg1(xh_hbm, aap_hbm, srcp_hbm, prow_hbm, tb_hbm,
          acc_hbm, den_hbm,
          tbbuf, prowbuf, adwin, srcbuf, rowbuf, asbuf, accbuf, dnbuf,
          sem0, sem1):
    def _sread(ref, idx):
        return ref[pl.ds(idx, 16)][0]

    wid = lax.axis_index("s") * jnp.int32(2) + lax.axis_index("c")
    pltpu.sync_copy(tb_hbm, tbbuf)
    n0 = _sread(tbbuf, wid)
    n1 = _sread(tbbuf, wid + jnp.int32(1))

    def refill_nodes(base):
        base = pl.multiple_of(base, 8)
        pltpu.sync_copy(prow_hbm.at[pl.ds(base, NODEWIN + 32)], prowbuf)
        pltpu.sync_copy(aap_hbm.at[pl.ds(base, NODEWIN)], adwin)

    nwb0 = (n0 // jnp.int32(8)) * jnp.int32(8)

    @pl.when(n1 > n0)
    def _():
        refill_nodes(nwb0)

    zero16 = jnp.zeros((16,), jnp.float32)
    for hg in range(4):
        dnbuf[pl.ds(hg * 16, 16)] = zero16
    for c in range(0, D1, 16):
        accbuf[pl.ds(c, 16)] = zero16

    def node_body(i, carry):
        nwb, swb = carry
        need_n = (i - nwb) >= jnp.int32(NODEWIN)

        @pl.when(need_n)
        def _():
            refill_nodes((i // jnp.int32(8)) * jnp.int32(8))

        nwb = jnp.where(need_n, (i // jnp.int32(8)) * jnp.int32(8), nwb)
        irel = i - nwb
        pcur = _sread(prowbuf, irel)
        pnext = _sread(prowbuf, irel + jnp.int32(1))
        a0 = (pcur // jnp.int32(8)) * jnp.int32(8)
        ngrp = (pnext - a0 + jnp.int32(7)) // jnp.int32(8)

        adrow = [adwin[irel, pl.ds(H1 + hg * 16, 16)] for hg in range(4)]

        def grp_body(g, swb):
            cur = pl.multiple_of(a0 + g * jnp.int32(8), 8)
            need_s = (cur - swb) + jnp.int32(8) > jnp.int32(SRCWIN)

            @pl.when(need_s)
            def _():
                pltpu.sync_copy(srcp_hbm.at[pl.ds(cur, SRCWIN)],
                                srcbuf.at[pl.ds(0, SRCWIN)])

            swb = jnp.where(need_s, cur, swb)
            rel = pl.multiple_of(cur - swb, 8)
            idxref = srcbuf.at[pl.ds(rel, 8)]
            cp0 = pltpu.async_copy(xh_hbm.at[idxref], rowbuf, sem0)
            cp1 = pltpu.async_copy(aap_hbm.at[idxref], asbuf, sem1)
            cp0.wait()
            cp1.wait()

            pvecs = []
            for j in range(8):
                t = cur + jnp.int32(j)
                w = jnp.where((t >= pcur) & (t < pnext), jnp.float32(1.0), jnp.float32(0.0))
                pj = []
                for hg in range(4):
                    e = asbuf[j, pl.ds(hg * 16, 16)] + adrow[hg]
                    e = jnp.maximum(e, 0.2 * e)
                    p = jnp.exp(e) * w
                    off = hg * 16
                    dnbuf[pl.ds(off, 16)] = dnbuf[pl.ds(off, 16)] + p
                    pj.append(p)
                pvecs.append(pj)

            def mac_body(c, z):
                base = c * jnp.int32(H1)
                for hg in range(4):
                    off = base + jnp.int32(hg * 16)
                    a = accbuf[pl.ds(off, 16)]
                    for j in range(8):
                        a = a + rowbuf[j, pl.ds(off, 16)] * pvecs[j][hg]
                    accbuf[pl.ds(off, 16)] = a
                return z

            lax.fori_loop(jnp.int32(0), jnp.int32(C1), mac_body, jnp.int32(0))
            return swb

        swb = lax.fori_loop(jnp.int32(0), ngrp, grp_body, swb)

        pltpu.sync_copy(accbuf, acc_hbm.at[i])
        pltpu.sync_copy(dnbuf, den_hbm.at[i])
        for hg in range(4):
            dnbuf[pl.ds(hg * 16, 16)] = zero16
        for c in range(0, D1, 16):
            accbuf[pl.ds(c, 16)] = zero16
        return (nwb, swb)

    lax.fori_loop(n0, n1, node_body, (nwb0, jnp.int32(-(1 << 28))))


_BLK_C = 200


def _mm2_body(acc_ref, den_ref, w2_ref, b1_ref, out_ref):
    dinv = 1.0 / den_ref[...]
    h = acc_ref[...].reshape(_BLK_C, C1, H1) * dinv[:, None, :]
    h = h + b1_ref[...][None]
    h = jnp.where(h > 0, h, jnp.exp(h) - 1.0)
    out_ref[...] = jnp.dot(h.reshape(_BLK_C, D1), w2_ref[...],
                           preferred_element_type=jnp.float32)


def _mm2(acc, den, w2pp, b1p):
    grid = (NN // _BLK_C,)
    return pl.pallas_call(
        _mm2_body,
        grid=grid,
        in_specs=[
            pl.BlockSpec((_BLK_C, D1), lambda i: (i, jnp.int32(0))),
            pl.BlockSpec((_BLK_C, H1), lambda i: (i, jnp.int32(0))),
            pl.BlockSpec((D1, 128), lambda i: (jnp.int32(0), jnp.int32(0))),
            pl.BlockSpec((C1, H1), lambda i: (jnp.int32(0), jnp.int32(0))),
        ],
        out_specs=pl.BlockSpec((_BLK_C, 128), lambda i: (i, jnp.int32(0))),
        out_shape=jax.ShapeDtypeStruct((NN, 128), jnp.float32),
    )(acc, den, w2pp, b1p)


@functools.partial(
    pl.kernel,
    mesh=_MESH,
    out_type=jax.ShapeDtypeStruct((NTILES, 16), jnp.float32),
    scratch_types=[
        pltpu.VMEM((56,), jnp.int32),
        pltpu.VMEM((NODEWIN + 32,), jnp.int32),
        pltpu.VMEM((NODEWIN, 128), jnp.float32),
        pltpu.VMEM((SRCWIN + 64,), jnp.int32),
        pltpu.VMEM((8, 128), jnp.float32),
        pltpu.VMEM((16,), jnp.float32),
        pltpu.VMEM((16,), jnp.float32),
        pltpu.VMEM((16,), jnp.float32),
        pltpu.SemaphoreType.DMA,
    ],
)
def _agg2(xw_hbm, srcp_hbm, prow_hbm, tb_hbm, a2s_hbm, a2d_hbm,
          out_hbm,
          tbbuf, prowbuf, xwwin, srcbuf, rowbuf, a2sbuf, a2dbuf,
          tsbuf, sem0):
    def _sread(ref, idx):
        return ref[pl.ds(idx, 16)][0]

    wid = lax.axis_index("s") * jnp.int32(2) + lax.axis_index("c")
    pltpu.sync_copy(tb_hbm, tbbuf)
    pltpu.sync_copy(a2s_hbm, a2sbuf)
    pltpu.sync_copy(a2d_hbm, a2dbuf)
    n0 = _sread(tbbuf, wid)
    n1 = _sread(tbbuf, wid + jnp.int32(1))
    a2s = a2sbuf[...]
    a2d = a2dbuf[...]
    colmask = lax.iota(jnp.int32, 16) < H2

    def refill_nodes(base):
        base = pl.multiple_of(base, 8)
        pltpu.sync_copy(prow_hbm.at[pl.ds(base, NODEWIN + 32)], prowbuf)
        pltpu.sync_copy(xw_hbm.at[pl.ds(base, NODEWIN)], xwwin)

    nwb0 = (n0 // jnp.int32(8)) * jnp.int32(8)

    @pl.when(n1 > n0)
    def _():
        refill_nodes(nwb0)

    tsbuf[...] = jnp.zeros((16,), jnp.float32)

    def node_body(i, carry):
        nwb, swb = carry
        need_n = (i - nwb) >= jnp.int32(NODEWIN)

        @pl.when(need_n)
        def _():
            refill_nodes((i // jnp.int32(8)) * jnp.int32(8))

        nwb = jnp.where(need_n, (i // jnp.int32(8)) * jnp.int32(8), nwb)
        irel = i - nwb
        pcur = _sread(prowbuf, irel)
        pnext = _sread(prowbuf, irel + jnp.int32(1))
        a0 = (pcur // jnp.int32(8)) * jnp.int32(8)
        ngrp = (pnext - a0 + jnp.int32(7)) // jnp.int32(8)
        adrow = xwwin[irel, pl.ds(0, 16)] * a2d

        def grp_body(g, carry2):
            swb, acc, dsum = carry2
            cur = pl.multiple_of(a0 + g * jnp.int32(8), 8)
            need_s = (cur - swb) + jnp.int32(8) > jnp.int32(SRCWIN)

            @pl.when(need_s)
            def _():
                pltpu.sync_copy(srcp_hbm.at[pl.ds(cur, SRCWIN)],
                                srcbuf.at[pl.ds(0, SRCWIN)])

            swb = jnp.where(need_s, cur, swb)
            rel = pl.multiple_of(cur - swb, 8)
            pltpu.async_copy(xw_hbm.at[srcbuf.at[pl.ds(rel, 8)]], rowbuf,
                             sem0).wait()
            for j in range(8):
                t = cur + jnp.int32(j)
                w = jnp.where((t >= pcur) & (t < pnext), jnp.float32(1.0), jnp.float32(0.0))
                srow = rowbuf[j, pl.ds(0, 16)]
                e = srow * a2s + adrow
                e = jnp.maximum(e, 0.2 * e)
                p = jnp.where(colmask, jnp.exp(e) * w, 0.0)
                acc = acc + p * srow
                dsum = dsum + p
            return (swb, acc, dsum)

        z16 = jnp.zeros((16,), jnp.float32)
        swb, acc, dsum = lax.fori_loop(jnp.int32(0), ngrp, grp_body, (swb, z16, z16))
        contrib = jnp.where(colmask, acc / dsum, 0.0)
        tsbuf[...] = tsbuf[...] + contrib
        return (nwb, swb)

    lax.fori_loop(n0, n1, node_body, (nwb0, jnp.int32(-(1 << 28))))
    pltpu.sync_copy(tsbuf, out_hbm.at[wid])


def _fin_body(ts_ref, b2_ref, out_ref):
    s = jnp.sum(ts_ref[...]) / (H2 * NN) + jnp.sum(b2_ref[...])
    out_ref[...] = jnp.broadcast_to(jnp.tanh(s), (1, 1))


def _fin(tilesums, b2):
    return pl.pallas_call(
        _fin_body,
        grid=(1,),
        in_specs=[
            pl.BlockSpec((NTILES, 16), lambda i: (jnp.int32(0), jnp.int32(0))),
            pl.BlockSpec((1, 1), lambda i: (jnp.int32(0), jnp.int32(0))),
        ],
        out_specs=pl.BlockSpec((1, 1), lambda i: (jnp.int32(0), jnp.int32(0))),
        out_shape=jax.ShapeDtypeStruct((1, 1), jnp.float32),
    )(tilesums, b2)


def kernel(x, edge_index, W1, att_src1, att_dst1, b1, W2, att_src2, att_dst2, b2):
    f32 = jnp.float32
    i32 = jnp.int32
    x = x.astype(f32)
    W1 = W1.astype(f32)
    W2 = W2.astype(f32)

    w1r = W1.reshape(D_IN, H1, C1)
    w1p = jnp.transpose(w1r, (0, 2, 1)).reshape(D_IN, D1)
    a_src_t = jnp.transpose(att_src1[0].astype(f32))
    a_dst_t = jnp.transpose(att_dst1[0].astype(f32))
    b1p = jnp.transpose(b1.astype(f32).reshape(H1, C1))
    w2p = jnp.transpose(W2.reshape(H1, C1, H2), (1, 0, 2)).reshape(D1, H2)
    w2pp = jnp.zeros((D1, 128), f32).at[:, :H2].set(w2p)
    a2s = jnp.zeros((16,), f32).at[:H2].set(att_src2[0, :, 0].astype(f32))
    a2d = jnp.zeros((16,), f32).at[:H2].set(att_dst2[0, :, 0].astype(f32))
    b2r = b2.astype(f32).reshape(1, 1)

    srce = edge_index[0].astype(i32)
    dste = edge_index[1].astype(i32)
    loop = jnp.arange(NN, dtype=i32)
    src_all = jnp.concatenate([srce, loop])
    dst_all = jnp.concatenate([dste, loop])
    order = jnp.argsort(dst_all)
    src_s = src_all[order]
    dst_s = dst_all[order]
    counts = jnp.zeros((NN,), i32).at[dst_all].add(1)
    rowptr = jnp.concatenate(
        [jnp.zeros((1,), i32), jnp.cumsum(counts).astype(i32)])
    srcp = jnp.concatenate([src_s, jnp.zeros((EPAD - EPRIME,), i32)])
    q = (rowptr[:-1] * NTILES) // EPRIME
    tbc = jnp.sum(
        (q[:, None] == jnp.arange(NTILES, dtype=i32)[None, :]), axis=0
    ).astype(i32)
    tb = jnp.concatenate([jnp.zeros((1,), i32), jnp.cumsum(tbc).astype(i32)])
    tb = jnp.concatenate([tb, jnp.zeros((23,), i32)])
    prow_p = jnp.concatenate(
        [rowptr, jnp.full((NPADROWS - NN - 1,), rowptr[NN], i32)])

    xh1, asad = _mm1(x, w1p, a_src_t, a_dst_t)
    aap = jnp.concatenate([asad, jnp.zeros((NPADROWS - NN, 2 * H1), f32)])

    acc, den = _agg1(xh1, aap, srcp, prow_p, tb)

    xh2 = _mm2(acc, den, w2pp, b1p)
    xh2p = jnp.concatenate([xh2, jnp.zeros((NPADROWS - NN, 128), f32)])

    tilesums = _agg2(xh2p, srcp, prow_p, tb, a2s, a2d)

    out = _fin(tilesums, b2r)
    return out.reshape(1)
